# Initial kernel scaffold; baseline (speedup 1.0000x reference)
import jax
import jax.numpy as jnp
from jax.experimental import pallas as pl


def kernel(length_q, length_k, embeddings_table):
    raise NotImplementedError("write your pallas kernel here")


# speedup vs baseline: 5.6446x; 5.6446x over previous
import functools

import jax
import jax.numpy as jnp
from jax import lax
from jax.experimental import pallas as pl
from jax.experimental.pallas import tpu as pltpu
from jax.experimental.pallas import tpu_sc as plsc

EMBED_DIM = 64
MAX_REL_POS = 128
LENGTH_Q = 2048
LENGTH_K = 2048

NUM_CORES = 2
NUM_SUBCORES = 16
NUM_WORKERS = NUM_CORES * NUM_SUBCORES

PAIR_W = 2 * EMBED_DIM
BAND_PAIRS = 2048
ROW_PAIRS = LENGTH_K * EMBED_DIM // PAIR_W
CHUNK = 128
ROWS_PER_WORKER = LENGTH_Q // NUM_WORKERS


def _sc_band_kernel(pairs_hbm, idx_hbm, out_hbm, idx_v, rows_v,
                    band_even, band_odd, sem):
    cid = lax.axis_index("c")
    sid = lax.axis_index("s")

    for half, band in ((0, band_even), (1, band_odd)):
        c = half * NUM_SUBCORES + sid
        pltpu.sync_copy(idx_hbm.at[pl.ds(c * CHUNK, CHUNK)], idx_v)
        pltpu.async_copy(pairs_hbm.at[idx_v], rows_v, sem).wait()
        pltpu.sync_copy(rows_v, band.at[pl.ds(sid * CHUNK, CHUNK)])
    plsc.subcore_barrier()

    wid = sid * NUM_CORES + cid
    for r in range(ROWS_PER_WORKER):
        i = wid * ROWS_PER_WORKER + r
        if r % 2 == 0:
            src = band_odd.at[pl.ds(((LENGTH_Q - 2) - i) // 2, ROW_PAIRS)]
        else:
            src = band_even.at[pl.ds(((LENGTH_Q - 1) - i) // 2, ROW_PAIRS)]
        pltpu.sync_copy(src, out_hbm.at[pl.ds(i * ROW_PAIRS, ROW_PAIRS)])


@functools.partial(
    pl.kernel,
    out_type=jax.ShapeDtypeStruct((LENGTH_Q * ROW_PAIRS, PAIR_W), jnp.float32),
    mesh=plsc.VectorSubcoreMesh(core_axis_name="c", subcore_axis_name="s"),
    scratch_types=[
        pltpu.VMEM((CHUNK,), jnp.int32),
        pltpu.VMEM((CHUNK, PAIR_W), jnp.float32),
        pltpu.VMEM_SHARED((BAND_PAIRS, PAIR_W), jnp.float32),
        pltpu.VMEM_SHARED((BAND_PAIRS, PAIR_W), jnp.float32),
        pltpu.SemaphoreType.DMA,
    ],
)
def _band_expand_and_replicate(pairs_hbm, idx_hbm, out_hbm, *scratch):
    _sc_band_kernel(pairs_hbm, idx_hbm, out_hbm, *scratch)


def kernel(length_q, length_k, embeddings_table):
    d0 = length_k - length_q
    m = jnp.arange(2 * BAND_PAIRS + 2)
    idx = (jnp.clip(m + d0 - (LENGTH_Q - 1), -MAX_REL_POS, MAX_REL_POS)
           + MAX_REL_POS)
    def pair_ids(a, b):
        return jnp.where(a == b, jnp.where(a == 0, 0, 257), a + 1)
    even_ids = pair_ids(idx[0:4096:2], idx[1:4096:2])
    odd_ids = pair_ids(idx[1:4097:2], idx[2:4098:2])
    pair_idx = jnp.concatenate([even_ids, odd_ids]).astype(jnp.int32)
    t0 = jnp.concatenate([embeddings_table[:1], embeddings_table[:1]], axis=1)
    mid = jnp.concatenate([embeddings_table[:-1], embeddings_table[1:]], axis=1)
    t256 = jnp.concatenate([embeddings_table[-1:], embeddings_table[-1:]],
                           axis=1)
    pair_table = jnp.concatenate([t0, mid, t256], axis=0)
    out2d = _band_expand_and_replicate(pair_table, pair_idx)
    return out2d.reshape(LENGTH_Q, LENGTH_K, EMBED_DIM)


# speedup vs baseline: 5.6473x; 1.0005x over previous
import functools

import jax
import jax.numpy as jnp
from jax import lax
from jax.experimental import pallas as pl
from jax.experimental.pallas import tpu as pltpu
from jax.experimental.pallas import tpu_sc as plsc

EMBED_DIM = 64
MAX_REL_POS = 128
LENGTH_Q = 2048
LENGTH_K = 2048

NUM_CORES = 2
NUM_SUBCORES = 16
NUM_WORKERS = NUM_CORES * NUM_SUBCORES

PAIR_W = 2 * EMBED_DIM
BAND_PAIRS = 2048
ROW_PAIRS = LENGTH_K * EMBED_DIM // PAIR_W
CHUNK = 128
ROWS_PER_WORKER = LENGTH_Q // NUM_WORKERS


def _sc_band_kernel(pairs_hbm, idx_hbm, out_hbm, idx_v, rows_v,
                    band_even, band_odd, sem):
    cid = lax.axis_index("c")
    sid = lax.axis_index("s")

    for half, band in ((0, band_even), (1, band_odd)):
        c = half * NUM_SUBCORES + sid
        pltpu.sync_copy(idx_hbm.at[pl.ds(c * CHUNK, CHUNK)], idx_v)
        pltpu.async_copy(pairs_hbm.at[idx_v], rows_v, sem).wait()
        pltpu.sync_copy(rows_v, band.at[pl.ds(sid * CHUNK, CHUNK)])
    plsc.subcore_barrier()

    wid = sid * NUM_CORES + cid
    copies = []
    for r in range(ROWS_PER_WORKER):
        i = wid * ROWS_PER_WORKER + r
        if r % 2 == 0:
            src = band_odd.at[pl.ds(((LENGTH_Q - 2) - i) // 2, ROW_PAIRS)]
        else:
            src = band_even.at[pl.ds(((LENGTH_Q - 1) - i) // 2, ROW_PAIRS)]
        copies.append(
            pltpu.async_copy(src, out_hbm.at[pl.ds(i * ROW_PAIRS, ROW_PAIRS)],
                             sem))
    for cp in copies:
        cp.wait()


@functools.partial(
    pl.kernel,
    out_type=jax.ShapeDtypeStruct((LENGTH_Q * ROW_PAIRS, PAIR_W), jnp.float32),
    mesh=plsc.VectorSubcoreMesh(core_axis_name="c", subcore_axis_name="s"),
    scratch_types=[
        pltpu.VMEM((CHUNK,), jnp.int32),
        pltpu.VMEM((CHUNK, PAIR_W), jnp.float32),
        pltpu.VMEM_SHARED((BAND_PAIRS, PAIR_W), jnp.float32),
        pltpu.VMEM_SHARED((BAND_PAIRS, PAIR_W), jnp.float32),
        pltpu.SemaphoreType.DMA,
    ],
)
def _band_expand_and_replicate(pairs_hbm, idx_hbm, out_hbm, *scratch):
    _sc_band_kernel(pairs_hbm, idx_hbm, out_hbm, *scratch)


def kernel(length_q, length_k, embeddings_table):
    d0 = length_k - length_q
    m = jnp.arange(2 * BAND_PAIRS + 2)
    idx = (jnp.clip(m + d0 - (LENGTH_Q - 1), -MAX_REL_POS, MAX_REL_POS)
           + MAX_REL_POS)
    def pair_ids(a, b):
        return jnp.where(a == b, jnp.where(a == 0, 0, 257), a + 1)
    even_ids = pair_ids(idx[0:4096:2], idx[1:4096:2])
    odd_ids = pair_ids(idx[1:4097:2], idx[2:4098:2])
    pair_idx = jnp.concatenate([even_ids, odd_ids]).astype(jnp.int32)
    t0 = jnp.concatenate([embeddings_table[:1], embeddings_table[:1]], axis=1)
    mid = jnp.concatenate([embeddings_table[:-1], embeddings_table[1:]], axis=1)
    t256 = jnp.concatenate([embeddings_table[-1:], embeddings_table[-1:]],
                           axis=1)
    pair_table = jnp.concatenate([t0, mid, t256], axis=0)
    out2d = _band_expand_and_replicate(pair_table, pair_idx)
    return out2d.reshape(LENGTH_Q, LENGTH_K, EMBED_DIM)


# speedup vs baseline: 6.6155x; 1.1714x over previous
import functools

import jax
import jax.numpy as jnp
from jax import lax
from jax.experimental import pallas as pl
from jax.experimental.pallas import tpu as pltpu
from jax.experimental.pallas import tpu_sc as plsc

EMBED_DIM = 64
MAX_REL_POS = 128
LENGTH_Q = 2048
LENGTH_K = 2048

NUM_CORES = 2
NUM_SUBCORES = 16
NUM_WORKERS = NUM_CORES * NUM_SUBCORES

PAIR_W = 2 * EMBED_DIM
BAND_PAIRS = 2048
BAND_ROWS = 2 * BAND_PAIRS
CHUNK = 128
LANES = 16
ROWS_PER_WORKER = LENGTH_Q // NUM_WORKERS


def _sc_band_kernel(pairs_hbm, idx_hbm, out_hbm, idx_v, rows_v, repack_v,
                    band_sh, sem):
    cid = lax.axis_index("c")
    sid = lax.axis_index("s")

    pltpu.sync_copy(idx_hbm.at[pl.ds(sid * CHUNK, CHUNK)], idx_v)
    pltpu.async_copy(pairs_hbm.at[idx_v], rows_v, sem).wait()
    for p in range(CHUNK):
        for k in range(PAIR_W // LANES):
            v = rows_v[p, pl.ds(k * LANES, LANES)]
            repack_v[2 * p + k // 4, pl.ds((k % 4) * LANES, LANES)] = v
    pltpu.sync_copy(repack_v, band_sh.at[pl.ds(sid * 2 * CHUNK, 2 * CHUNK)])
    plsc.subcore_barrier()

    wid = sid * NUM_CORES + cid
    copies = []
    for r in range(ROWS_PER_WORKER):
        i = wid * ROWS_PER_WORKER + r
        copies.append(
            pltpu.async_copy(band_sh.at[pl.ds((LENGTH_Q - 1) - i, LENGTH_K)],
                             out_hbm.at[i], sem))
    for cp in copies:
        cp.wait()


@functools.partial(
    pl.kernel,
    out_type=jax.ShapeDtypeStruct((LENGTH_Q, LENGTH_K, EMBED_DIM),
                                  jnp.float32),
    mesh=plsc.VectorSubcoreMesh(core_axis_name="c", subcore_axis_name="s"),
    scratch_types=[
        pltpu.VMEM((CHUNK,), jnp.int32),
        pltpu.VMEM((CHUNK, PAIR_W), jnp.float32),
        pltpu.VMEM((2 * CHUNK, EMBED_DIM), jnp.float32),
        pltpu.VMEM_SHARED((BAND_ROWS, EMBED_DIM), jnp.float32),
        pltpu.SemaphoreType.DMA,
    ],
)
def _band_expand_and_replicate(pairs_hbm, idx_hbm, out_hbm, *scratch):
    _sc_band_kernel(pairs_hbm, idx_hbm, out_hbm, *scratch)


def kernel(length_q, length_k, embeddings_table):
    d0 = length_k - length_q
    m = jnp.arange(BAND_ROWS)
    idx = (jnp.clip(m + d0 - (LENGTH_Q - 1), -MAX_REL_POS, MAX_REL_POS)
           + MAX_REL_POS)
    a, b = idx[0::2], idx[1::2]
    pair_idx = jnp.where(a == b, jnp.where(a == 0, 0, 257), a + 1)
    pair_idx = pair_idx.astype(jnp.int32)
    t0 = jnp.concatenate([embeddings_table[:1], embeddings_table[:1]], axis=1)
    mid = jnp.concatenate([embeddings_table[:-1], embeddings_table[1:]], axis=1)
    t256 = jnp.concatenate([embeddings_table[-1:], embeddings_table[-1:]],
                           axis=1)
    pair_table = jnp.concatenate([t0, mid, t256], axis=0)
    return _band_expand_and_replicate(pair_table, pair_idx)
